# baseline (speedup 1.0000x reference)
import jax
import jax.numpy as jnp
from jax import lax
from jax.experimental import pallas as pl
from jax.experimental.pallas import tpu as pltpu
from jax.experimental.pallas import tpu_sc as plsc

N = 10000
E = 320000
D = 128
H = 64
EPS = 1e-5

NSUB = 16
EPT = E // NSUB
CHUNK = 80
PK = 2 * CHUNK
NCH = EPT // CHUNK
NG = NCH // 2
TILES_IO = 10
RPT = N // TILES_IO
ZR = 8


def _matmul_body(h_ref, w_ref, zlo_ref, zhi_ref):
    z = lax.dot_general(
        h_ref[...], w_ref[...], (((1,), (1,)), ((), ())),
        preferred_element_type=jnp.float32, precision=lax.Precision.HIGHEST)
    zlo_ref[...] = z[:, 0:H]
    zhi_ref[...] = z[:, H:D]


def _edge_body(zlo_hbm, zhi_hbm, pidx_hbm, out_hbm,
               acc_sh, zbuf,
               pib0, pib1, sci0, sci1,
               zsd0, zsd1, ct0, ct1,
               sio0, sio1, sg0, sg1, ssc0, ssc1):
    c = lax.axis_index("c")
    s = lax.axis_index("s")

    pib = (pib0, pib1)
    sci = (sci0, sci1)
    zsd = (zsd0, zsd1)
    ct = (ct0, ct1)
    sio = (sio0, sio1)
    sg = (sg0, sg1)
    ssc = (ssc0, ssc1)

    def issue_idx(i, b):
        pltpu.async_copy(pidx_hbm.at[pl.ds((s * NCH + i) * PK, PK)],
                         pib[b], sio[b])

    def wait_idx(b):
        pltpu.make_async_copy(pidx_hbm.at[pl.ds(0, PK)], pib[b], sio[b]).wait()

    def issue_gather(b):
        @pl.when(c == 0)
        def _lo():
            pltpu.async_copy(zlo_hbm.at[pib[b]], zsd[b], sg[b])

        @pl.when(c == 1)
        def _hi():
            pltpu.async_copy(zhi_hbm.at[pib[b]], zsd[b], sg[b])

    def wait_gather(b):
        pltpu.make_async_copy(zlo_hbm.at[pib[b]], zsd[b], sg[b]).wait()

    def wait_scatter(b):
        pltpu.make_async_copy(ct[b], acc_sh.at[sci[b]], ssc[b]).wait()

    issue_idx(0, 0)
    issue_idx(1, 1)

    @pl.when(s < TILES_IO)
    def _zero_acc():
        def zfill(r, carry):
            for k in range(D // 16):
                zbuf[r, pl.ds(16 * k, 16)] = jnp.zeros((16,), jnp.float32)
            return carry

        lax.fori_loop(0, ZR, zfill, 0)

        def zcopy(j, carry):
            pltpu.sync_copy(zbuf, acc_sh.at[pl.ds(s * RPT + j * ZR, ZR)])
            return carry

        lax.fori_loop(0, RPT // ZR, zcopy, 0)

    wait_idx(0)
    issue_gather(0)
    plsc.subcore_barrier()

    def gloop(g, carry):
        for b in range(2):
            i = g * 2 + b
            bn = 1 - b

            @pl.when(i + 1 < NCH)
            def _advance():
                wait_idx(bn)
                issue_gather(bn)

            wait_gather(b)

            @pl.when(i >= 2)
            def _drain_scatter():
                wait_scatter(b)

            for k in range(CHUNK // 16):
                sci[b][pl.ds(16 * k, 16)] = pib[b][pl.ds(CHUNK + 16 * k, 16)]

            @pl.when(i + 2 < NCH)
            def _next_idx():
                issue_idx(i + 2, b)

            @plsc.parallel_loop(0, CHUNK, step=1, unroll=4)
            def _rows(r):
                for k in range(H // 16):
                    sv = zsd[b][r, pl.ds(16 * k, 16)]
                    dv = zsd[b][CHUNK + r, pl.ds(16 * k, 16)]
                    ee = jnp.exp(sv * dv)
                    ct[b][r, pl.ds(16 * k, 16)] = ee
                    ct[b][r, pl.ds(H + 16 * k, 16)] = ee * sv

            pltpu.async_copy(ct[b], acc_sh.at[sci[b]], ssc[b], add=True)
        return carry

    lax.fori_loop(0, NG, gloop, 0)
    for b in range(2):
        wait_scatter(b)
    plsc.subcore_barrier()

    @pl.when(s < TILES_IO)
    def _dump_acc():
        pltpu.sync_copy(acc_sh.at[pl.ds(s * RPT, RPT)],
                        out_hbm.at[pl.ds(c * N + s * RPT, RPT)])


_edge_kernel = pl.kernel(
    _edge_body,
    out_type=jax.ShapeDtypeStruct((2 * N, D), jnp.float32),
    mesh=plsc.VectorSubcoreMesh(core_axis_name="c", subcore_axis_name="s"),
    scratch_types=[
        pltpu.VMEM_SHARED((N, D), jnp.float32),
        pltpu.VMEM((ZR, D), jnp.float32),
        pltpu.VMEM((PK,), jnp.int32),
        pltpu.VMEM((PK,), jnp.int32),
        pltpu.VMEM((CHUNK,), jnp.int32),
        pltpu.VMEM((CHUNK,), jnp.int32),
        pltpu.VMEM((PK, H), jnp.float32),
        pltpu.VMEM((PK, H), jnp.float32),
        pltpu.VMEM((CHUNK, D), jnp.float32),
        pltpu.VMEM((CHUNK, D), jnp.float32),
        pltpu.SemaphoreType.DMA,
        pltpu.SemaphoreType.DMA,
        pltpu.SemaphoreType.DMA,
        pltpu.SemaphoreType.DMA,
        pltpu.SemaphoreType.DMA,
        pltpu.SemaphoreType.DMA,
    ],
    compiler_params=pltpu.CompilerParams(use_tc_tiling_on_sc=False),
)


def _post_body(acc_ref, snorm_ref, gamma_ref, beta_ref, out_ref):
    sn = snorm_ref[...]
    for hh in range(2):
        dnm = acc_ref[hh * N:(hh + 1) * N, 0:H]
        num = acc_ref[hh * N:(hh + 1) * N, H:D]
        dnm = jnp.where(dnm == 0.0, 1.0, dnm)
        hagg = num / dnm * sn
        mu = jnp.mean(hagg, axis=0, keepdims=True)
        xc = hagg - mu
        var = jnp.mean(xc * xc, axis=0, keepdims=True)
        g = gamma_ref[0:1, hh * H:(hh + 1) * H]
        b = beta_ref[0:1, hh * H:(hh + 1) * H]
        y = xc * (g * lax.rsqrt(var + EPS)) + b
        out_ref[:, hh * H:(hh + 1) * H] = jnp.where(y > 0, y, jnp.exp(y) - 1.0)


def kernel(h, edge_index, snorm_n, W_fc, gamma, beta):
    src = edge_index[0].astype(jnp.int32)
    dst = edge_index[1].astype(jnp.int32)
    pidx = jnp.concatenate(
        [src.reshape(-1, CHUNK), dst.reshape(-1, CHUNK)], axis=1).reshape(-1)
    z_lo, z_hi = pl.pallas_call(
        _matmul_body,
        out_shape=[jax.ShapeDtypeStruct((N, H), jnp.float32),
                   jax.ShapeDtypeStruct((N, H), jnp.float32)],
    )(h, W_fc)
    acc = _edge_kernel(z_lo, z_hi, pidx)
    out = pl.pallas_call(
        _post_body,
        out_shape=jax.ShapeDtypeStruct((N, D), jnp.float32),
    )(acc, snorm_n, gamma.reshape(1, D), beta.reshape(1, D))
    return out


# speedup vs baseline: 1.1109x; 1.1109x over previous
import numpy as np
import jax
import jax.numpy as jnp
from jax import lax
from jax.experimental import pallas as pl
from jax.experimental.pallas import tpu as pltpu
from jax.experimental.pallas import tpu_sc as plsc

N = 10000
E = 320000
D = 128
H = 64
EPS = 1e-5

NSUB = 16
EPT = E // NSUB
CHUNK = 80
PK = 2 * CHUNK
NCH = EPT // CHUNK
NG = NCH // 2
TILES_IO = 10
RPT = N // TILES_IO
ZR = 8

_t = np.arange(H)
_PERM = (_t // 32) * 32 + (_t % 2) * 16 + (_t % 32) // 2


def _matmul_body(h_ref, w_ref, z_ref):
    z_ref[...] = lax.dot_general(
        h_ref[...], w_ref[...], (((1,), (1,)), ((), ())),
        preferred_element_type=jnp.float32, precision=lax.Precision.HIGHEST)


def _edge_body(zlo_hbm, zhi_hbm, pidx_hbm, out_hbm,
               acc_sh, zbuf,
               pib0, pib1, sci0, sci1,
               zsd0, zsd1, ct0, ct1,
               sio0, sio1, sg0, sg1, ssc0, ssc1):
    c = lax.axis_index("c")
    s = lax.axis_index("s")

    pib = (pib0, pib1)
    sci = (sci0, sci1)
    zsd = (zsd0, zsd1)
    ct = (ct0, ct1)
    sio = (sio0, sio1)
    sg = (sg0, sg1)
    ssc = (ssc0, ssc1)

    def issue_idx(i, b):
        pltpu.async_copy(pidx_hbm.at[pl.ds((s * NCH + i) * PK, PK)],
                         pib[b], sio[b])

    def wait_idx(b):
        pltpu.make_async_copy(pidx_hbm.at[pl.ds(0, PK)], pib[b], sio[b]).wait()

    def issue_gather(b):
        @pl.when(c == 0)
        def _lo():
            pltpu.async_copy(zlo_hbm.at[pib[b]], zsd[b], sg[b])

        @pl.when(c == 1)
        def _hi():
            pltpu.async_copy(zhi_hbm.at[pib[b]], zsd[b], sg[b])

    def wait_gather(b):
        pltpu.make_async_copy(zlo_hbm.at[pib[b]], zsd[b], sg[b]).wait()

    def wait_scatter(b):
        pltpu.make_async_copy(ct[b], acc_sh.at[sci[b]], ssc[b]).wait()

    issue_idx(0, 0)
    issue_idx(1, 1)

    @pl.when(s < TILES_IO)
    def _zero_acc():
        def zfill(r, carry):
            for k in range(D // 16):
                zbuf[r, pl.ds(16 * k, 16)] = jnp.zeros((16,), jnp.float32)
            return carry

        lax.fori_loop(0, ZR, zfill, 0)

        def zcopy(j, carry):
            pltpu.sync_copy(zbuf, acc_sh.at[pl.ds(s * RPT + j * ZR, ZR)])
            return carry

        lax.fori_loop(0, RPT // ZR, zcopy, 0)

    wait_idx(0)
    issue_gather(0)
    plsc.subcore_barrier()

    hi_mask = jnp.full((16,), -65536, dtype=jnp.int32)

    def gloop(g, carry):
        for b in range(2):
            i = g * 2 + b
            bn = 1 - b

            @pl.when(i + 1 < NCH)
            def _advance():
                wait_idx(bn)
                issue_gather(bn)

            wait_gather(b)

            @pl.when(i >= 2)
            def _drain_scatter():
                wait_scatter(b)

            for k in range(CHUNK // 16):
                sci[b][pl.ds(16 * k, 16)] = pib[b][pl.ds(CHUNK + 16 * k, 16)]

            @pl.when(i + 2 < NCH)
            def _next_idx():
                issue_idx(i + 2, b)

            @plsc.parallel_loop(0, CHUNK, step=1, unroll=4)
            def _rows(r):
                for k2 in range(H // 32):
                    ws = plsc.bitcast(
                        zsd[b][r, pl.ds(32 * k2, 32)], jnp.int32)
                    wd = plsc.bitcast(
                        zsd[b][CHUNK + r, pl.ds(32 * k2, 32)], jnp.int32)
                    sv_lo = lax.bitcast_convert_type(
                        lax.shift_left(ws, 16), jnp.float32)
                    sv_hi = lax.bitcast_convert_type(
                        lax.bitwise_and(ws, hi_mask), jnp.float32)
                    dv_lo = lax.bitcast_convert_type(
                        lax.shift_left(wd, 16), jnp.float32)
                    dv_hi = lax.bitcast_convert_type(
                        lax.bitwise_and(wd, hi_mask), jnp.float32)
                    ee_lo = jnp.exp(sv_lo * dv_lo)
                    ee_hi = jnp.exp(sv_hi * dv_hi)
                    ct[b][r, pl.ds(32 * k2, 16)] = ee_lo
                    ct[b][r, pl.ds(32 * k2 + 16, 16)] = ee_hi
                    ct[b][r, pl.ds(H + 32 * k2, 16)] = ee_lo * sv_lo
                    ct[b][r, pl.ds(H + 32 * k2 + 16, 16)] = ee_hi * sv_hi

            pltpu.async_copy(ct[b], acc_sh.at[sci[b]], ssc[b], add=True)
        return carry

    lax.fori_loop(0, NG, gloop, 0)
    for b in range(2):
        wait_scatter(b)
    plsc.subcore_barrier()

    @pl.when(s < TILES_IO)
    def _dump_acc():
        pltpu.sync_copy(acc_sh.at[pl.ds(s * RPT, RPT)],
                        out_hbm.at[pl.ds(c * N + s * RPT, RPT)])


_edge_kernel = pl.kernel(
    _edge_body,
    out_type=jax.ShapeDtypeStruct((2 * N, D), jnp.float32),
    mesh=plsc.VectorSubcoreMesh(core_axis_name="c", subcore_axis_name="s"),
    scratch_types=[
        pltpu.VMEM_SHARED((N, D), jnp.float32),
        pltpu.VMEM((ZR, D), jnp.float32),
        pltpu.VMEM((PK,), jnp.int32),
        pltpu.VMEM((PK,), jnp.int32),
        pltpu.VMEM((CHUNK,), jnp.int32),
        pltpu.VMEM((CHUNK,), jnp.int32),
        pltpu.VMEM((PK, H), jnp.bfloat16),
        pltpu.VMEM((PK, H), jnp.bfloat16),
        pltpu.VMEM((CHUNK, D), jnp.float32),
        pltpu.VMEM((CHUNK, D), jnp.float32),
        pltpu.SemaphoreType.DMA,
        pltpu.SemaphoreType.DMA,
        pltpu.SemaphoreType.DMA,
        pltpu.SemaphoreType.DMA,
        pltpu.SemaphoreType.DMA,
        pltpu.SemaphoreType.DMA,
    ],
    compiler_params=pltpu.CompilerParams(use_tc_tiling_on_sc=False,
                                         needs_layout_passes=False),
)


def _post_body(acc_ref, snorm_ref, gamma_ref, beta_ref, out_ref):
    sn = snorm_ref[...]
    for hh in range(2):
        dnm = acc_ref[hh * N:(hh + 1) * N, 0:H]
        num = acc_ref[hh * N:(hh + 1) * N, H:D]
        dnm = jnp.where(dnm == 0.0, 1.0, dnm)
        hagg = num / dnm * sn
        mu = jnp.mean(hagg, axis=0, keepdims=True)
        xc = hagg - mu
        var = jnp.mean(xc * xc, axis=0, keepdims=True)
        g = gamma_ref[0:1, hh * H:(hh + 1) * H]
        b = beta_ref[0:1, hh * H:(hh + 1) * H]
        y = xc * (g * lax.rsqrt(var + EPS)) + b
        out_ref[:, hh * H:(hh + 1) * H] = jnp.where(y > 0, y, jnp.exp(y) - 1.0)


def kernel(h, edge_index, snorm_n, W_fc, gamma, beta):
    src = edge_index[0].astype(jnp.int32)
    dst = edge_index[1].astype(jnp.int32)
    pidx = jnp.concatenate(
        [src.reshape(-1, CHUNK), dst.reshape(-1, CHUNK)], axis=1).reshape(-1)
    z = pl.pallas_call(
        _matmul_body,
        out_shape=jax.ShapeDtypeStruct((N, D), jnp.float32),
    )(h, W_fc)
    z_lo = z[:, _PERM].astype(jnp.bfloat16)
    z_hi = z[:, H + _PERM].astype(jnp.bfloat16)
    acc = _edge_kernel(z_lo, z_hi, pidx)
    out = pl.pallas_call(
        _post_body,
        out_shape=jax.ShapeDtypeStruct((N, D), jnp.float32),
    )(acc, snorm_n, gamma.reshape(1, D), beta.reshape(1, D))
    return out


# speedup vs baseline: 1.1606x; 1.0447x over previous
import numpy as np
import jax
import jax.numpy as jnp
from jax import lax
from jax.experimental import pallas as pl
from jax.experimental.pallas import tpu as pltpu
from jax.experimental.pallas import tpu_sc as plsc

N = 10000
E = 320000
D = 128
H = 64
EPS = 1e-5

NSUB = 16
EPT = E // NSUB
CHUNK = 80
PK = 2 * CHUNK
NCH = EPT // CHUNK
NG = NCH // 2
TILES_IO = 10
RPT = N // TILES_IO
ZR = 8

_t = np.arange(H)
_PERM = (_t // 32) * 32 + (_t % 2) * 16 + (_t % 32) // 2


def _matmul_body(h_ref, w_ref, zlo_ref, zhi_ref):
    z = lax.dot_general(
        h_ref[...], w_ref[...], (((1,), (1,)), ((), ())),
        preferred_element_type=jnp.float32, precision=lax.Precision.HIGHEST)
    zlo_ref[...] = z[:, 0:H].astype(jnp.bfloat16)
    zhi_ref[...] = z[:, H:D].astype(jnp.bfloat16)


def _edge_body(zlo_hbm, zhi_hbm, pidx_hbm, out_hbm,
               acc_sh, zbuf,
               pib0, pib1, sci0, sci1,
               zsd0, zsd1, ct0, ct1,
               sio0, sio1, sg0, sg1, ssc0, ssc1):
    c = lax.axis_index("c")
    s = lax.axis_index("s")

    pib = (pib0, pib1)
    sci = (sci0, sci1)
    zsd = (zsd0, zsd1)
    ct = (ct0, ct1)
    sio = (sio0, sio1)
    sg = (sg0, sg1)
    ssc = (ssc0, ssc1)

    def issue_idx(i, b):
        pltpu.async_copy(pidx_hbm.at[pl.ds((s * NCH + i) * PK, PK)],
                         pib[b], sio[b])

    def wait_idx(b):
        pltpu.make_async_copy(pidx_hbm.at[pl.ds(0, PK)], pib[b], sio[b]).wait()

    def issue_gather(b):
        @pl.when(c == 0)
        def _lo():
            pltpu.async_copy(zlo_hbm.at[pib[b]], zsd[b], sg[b])

        @pl.when(c == 1)
        def _hi():
            pltpu.async_copy(zhi_hbm.at[pib[b]], zsd[b], sg[b])

    def wait_gather(b):
        pltpu.make_async_copy(zlo_hbm.at[pib[b]], zsd[b], sg[b]).wait()

    def wait_scatter(b):
        pltpu.make_async_copy(ct[b], acc_sh.at[sci[b]], ssc[b]).wait()

    issue_idx(0, 0)
    issue_idx(1, 1)

    @pl.when(s < TILES_IO)
    def _zero_acc():
        def zfill(r, carry):
            for k in range(D // 16):
                zbuf[r, pl.ds(16 * k, 16)] = jnp.zeros((16,), jnp.float32)
            return carry

        lax.fori_loop(0, ZR, zfill, 0)

        def zcopy(j, carry):
            pltpu.sync_copy(zbuf, acc_sh.at[pl.ds(s * RPT + j * ZR, ZR)])
            return carry

        lax.fori_loop(0, RPT // ZR, zcopy, 0)

    wait_idx(0)
    issue_gather(0)
    plsc.subcore_barrier()

    hi_mask = jnp.full((16,), -65536, dtype=jnp.int32)

    def gloop(g, carry):
        for b in range(2):
            i = g * 2 + b
            bn = 1 - b

            @pl.when(i + 1 < NCH)
            def _advance():
                wait_idx(bn)
                issue_gather(bn)

            wait_gather(b)

            @pl.when(i >= 2)
            def _drain_scatter():
                wait_scatter(b)

            for k in range(CHUNK // 16):
                sci[b][pl.ds(16 * k, 16)] = pib[b][pl.ds(CHUNK + 16 * k, 16)]

            @pl.when(i + 2 < NCH)
            def _next_idx():
                issue_idx(i + 2, b)

            @plsc.parallel_loop(0, CHUNK, step=1, unroll=4)
            def _rows(r):
                for k2 in range(H // 32):
                    ws = plsc.bitcast(
                        zsd[b][r, pl.ds(32 * k2, 32)], jnp.int32)
                    wd = plsc.bitcast(
                        zsd[b][CHUNK + r, pl.ds(32 * k2, 32)], jnp.int32)
                    sv_lo = lax.bitcast_convert_type(
                        lax.shift_left(ws, 16), jnp.float32)
                    sv_hi = lax.bitcast_convert_type(
                        lax.bitwise_and(ws, hi_mask), jnp.float32)
                    dv_lo = lax.bitcast_convert_type(
                        lax.shift_left(wd, 16), jnp.float32)
                    dv_hi = lax.bitcast_convert_type(
                        lax.bitwise_and(wd, hi_mask), jnp.float32)
                    ee_lo = jnp.exp(sv_lo * dv_lo)
                    ee_hi = jnp.exp(sv_hi * dv_hi)
                    ct[b][r, pl.ds(32 * k2, 16)] = ee_lo
                    ct[b][r, pl.ds(32 * k2 + 16, 16)] = ee_hi
                    ct[b][r, pl.ds(H + 32 * k2, 16)] = ee_lo * sv_lo
                    ct[b][r, pl.ds(H + 32 * k2 + 16, 16)] = ee_hi * sv_hi

            pltpu.async_copy(ct[b], acc_sh.at[sci[b]], ssc[b], add=True)
        return carry

    lax.fori_loop(0, NG, gloop, 0)
    for b in range(2):
        wait_scatter(b)
    plsc.subcore_barrier()

    @pl.when(s < TILES_IO)
    def _dump_acc():
        pltpu.sync_copy(acc_sh.at[pl.ds(s * RPT, RPT)],
                        out_hbm.at[pl.ds(c * N + s * RPT, RPT)])


_edge_kernel = pl.kernel(
    _edge_body,
    out_type=jax.ShapeDtypeStruct((2 * N, D), jnp.float32),
    mesh=plsc.VectorSubcoreMesh(core_axis_name="c", subcore_axis_name="s"),
    scratch_types=[
        pltpu.VMEM_SHARED((N, D), jnp.float32),
        pltpu.VMEM((ZR, D), jnp.float32),
        pltpu.VMEM((PK,), jnp.int32),
        pltpu.VMEM((PK,), jnp.int32),
        pltpu.VMEM((CHUNK,), jnp.int32),
        pltpu.VMEM((CHUNK,), jnp.int32),
        pltpu.VMEM((PK, H), jnp.bfloat16),
        pltpu.VMEM((PK, H), jnp.bfloat16),
        pltpu.VMEM((CHUNK, D), jnp.float32),
        pltpu.VMEM((CHUNK, D), jnp.float32),
        pltpu.SemaphoreType.DMA,
        pltpu.SemaphoreType.DMA,
        pltpu.SemaphoreType.DMA,
        pltpu.SemaphoreType.DMA,
        pltpu.SemaphoreType.DMA,
        pltpu.SemaphoreType.DMA,
    ],
    compiler_params=pltpu.CompilerParams(use_tc_tiling_on_sc=False,
                                         needs_layout_passes=False),
)


def _post_body(acc_ref, snorm_ref, gamma_ref, beta_ref, out_ref):
    sn = snorm_ref[...]
    for hh in range(2):
        dnm = acc_ref[hh * N:(hh + 1) * N, 0:H]
        num = acc_ref[hh * N:(hh + 1) * N, H:D]
        dnm = jnp.where(dnm == 0.0, 1.0, dnm)
        hagg = num / dnm * sn
        mu = jnp.mean(hagg, axis=0, keepdims=True)
        xc = hagg - mu
        var = jnp.mean(xc * xc, axis=0, keepdims=True)
        g = gamma_ref[0:1, hh * H:(hh + 1) * H]
        b = beta_ref[0:1, hh * H:(hh + 1) * H]
        y = xc * (g * lax.rsqrt(var + EPS)) + b
        out_ref[:, hh * H:(hh + 1) * H] = jnp.where(y > 0, y, jnp.exp(y) - 1.0)


def kernel(h, edge_index, snorm_n, W_fc, gamma, beta):
    src = edge_index[0].astype(jnp.int32)
    dst = edge_index[1].astype(jnp.int32)
    pidx = jnp.concatenate(
        [src.reshape(-1, CHUNK), dst.reshape(-1, CHUNK)], axis=1).reshape(-1)
    w_perm = W_fc[np.concatenate([_PERM, H + _PERM])]
    z_lo, z_hi = pl.pallas_call(
        _matmul_body,
        out_shape=[jax.ShapeDtypeStruct((N, H), jnp.bfloat16),
                   jax.ShapeDtypeStruct((N, H), jnp.bfloat16)],
    )(h, w_perm)
    acc = _edge_kernel(z_lo, z_hi, pidx)
    out = pl.pallas_call(
        _post_body,
        out_shape=jax.ShapeDtypeStruct((N, D), jnp.float32),
    )(acc, snorm_n, gamma.reshape(1, D), beta.reshape(1, D))
    return out


# speedup vs baseline: 1.2264x; 1.0567x over previous
import numpy as np
import jax
import jax.numpy as jnp
from jax import lax
from jax.experimental import pallas as pl
from jax.experimental.pallas import tpu as pltpu
from jax.experimental.pallas import tpu_sc as plsc

N = 10000
E = 320000
D = 128
H = 64
EPS = 1e-5

NSUB = 16
EPT = E // NSUB
CHUNK = 80
PK = 2 * CHUNK
NCH = EPT // CHUNK
NG = NCH // 2
TILES_IO = 10
RPT = N // TILES_IO
ZR = 8

_t = np.arange(H)
_PERM = (_t // 32) * 32 + (_t % 2) * 16 + (_t % 32) // 2


def _matmul_body(h_ref, w_ref, zlo_ref, zhi_ref):
    z = lax.dot_general(
        h_ref[...], w_ref[...], (((1,), (1,)), ((), ())),
        preferred_element_type=jnp.float32, precision=lax.Precision.HIGHEST)
    zlo_ref[...] = z[:, 0:H].astype(jnp.bfloat16)
    zhi_ref[...] = z[:, H:D].astype(jnp.bfloat16)


def _edge_body(zlo_hbm, zhi_hbm, pidx_hbm, out_hbm,
               acc_sh, zbuf,
               pib0, pib1, pib2, pib3, sci0, sci1,
               zsd0, zsd1, zsd2, zsd3, ct0, ct1,
               sio0, sio1, sio2, sio3, sg0, sg1, sg2, sg3, ssc0, ssc1):
    c = lax.axis_index("c")
    s = lax.axis_index("s")

    pib = (pib0, pib1, pib2, pib3)
    sci = (sci0, sci1)
    zsd = (zsd0, zsd1, zsd2, zsd3)
    ct = (ct0, ct1)
    sio = (sio0, sio1, sio2, sio3)
    sg = (sg0, sg1, sg2, sg3)
    ssc = (ssc0, ssc1)

    def issue_idx(i, b):
        pltpu.async_copy(pidx_hbm.at[pl.ds((s * NCH + i) * PK, PK)],
                         pib[b], sio[b])

    def wait_idx(b):
        pltpu.make_async_copy(pidx_hbm.at[pl.ds(0, PK)], pib[b], sio[b]).wait()

    def issue_gather(b):
        @pl.when(c == 0)
        def _lo():
            pltpu.async_copy(zlo_hbm.at[pib[b]], zsd[b], sg[b])

        @pl.when(c == 1)
        def _hi():
            pltpu.async_copy(zhi_hbm.at[pib[b]], zsd[b], sg[b])

    def wait_gather(b):
        pltpu.make_async_copy(zlo_hbm.at[pib[b]], zsd[b], sg[b]).wait()

    def wait_scatter(b):
        pltpu.make_async_copy(ct[b], acc_sh.at[sci[b]], ssc[b]).wait()

    for u in range(4):
        issue_idx(u, u)

    @pl.when(s < TILES_IO)
    def _zero_acc():
        def zfill(r, carry):
            for k in range(D // 16):
                zbuf[r, pl.ds(16 * k, 16)] = jnp.zeros((16,), jnp.float32)
            return carry

        lax.fori_loop(0, ZR, zfill, 0)

        def zcopy(j, carry):
            pltpu.sync_copy(zbuf, acc_sh.at[pl.ds(s * RPT + j * ZR, ZR)])
            return carry

        lax.fori_loop(0, RPT // ZR, zcopy, 0)

    wait_idx(0)
    issue_gather(0)
    wait_idx(1)
    issue_gather(1)
    plsc.subcore_barrier()

    hi_mask = jnp.full((16,), -65536, dtype=jnp.int32)

    def chunk_step(i, u, main):
        uc = u % 2
        wait_gather(u)

        def drain():
            wait_scatter(uc)

        if isinstance(i, int):
            if i >= 2:
                drain()
        else:
            pl.when(i >= 2)(drain)

        for k in range(CHUNK // 16):
            sci[uc][pl.ds(16 * k, 16)] = pib[u][pl.ds(CHUNK + 16 * k, 16)]

        if main:
            @pl.when(i + 4 < NCH)
            def _next_idx():
                issue_idx(i + 4, u)

            wait_idx((u + 2) % 4)
            issue_gather((u + 2) % 4)

        @plsc.parallel_loop(0, CHUNK, step=1, unroll=4)
        def _rows(r):
            for k2 in range(H // 32):
                ws = plsc.bitcast(
                    zsd[u][r, pl.ds(32 * k2, 32)], jnp.int32)
                wd = plsc.bitcast(
                    zsd[u][CHUNK + r, pl.ds(32 * k2, 32)], jnp.int32)
                sv_lo = lax.bitcast_convert_type(
                    lax.shift_left(ws, 16), jnp.float32)
                sv_hi = lax.bitcast_convert_type(
                    lax.bitwise_and(ws, hi_mask), jnp.float32)
                dv_lo = lax.bitcast_convert_type(
                    lax.shift_left(wd, 16), jnp.float32)
                dv_hi = lax.bitcast_convert_type(
                    lax.bitwise_and(wd, hi_mask), jnp.float32)
                ee_lo = jnp.exp(sv_lo * dv_lo)
                ee_hi = jnp.exp(sv_hi * dv_hi)
                ct[uc][r, pl.ds(32 * k2, 16)] = ee_lo
                ct[uc][r, pl.ds(32 * k2 + 16, 16)] = ee_hi
                ct[uc][r, pl.ds(H + 32 * k2, 16)] = ee_lo * sv_lo
                ct[uc][r, pl.ds(H + 32 * k2 + 16, 16)] = ee_hi * sv_hi

        pltpu.async_copy(ct[uc], acc_sh.at[sci[uc]], ssc[uc], add=True)

    def gloop(g, carry):
        for u in range(4):
            chunk_step(g * 4 + u, u, True)
        return carry

    lax.fori_loop(0, (NCH - 2) // 4, gloop, 0)
    chunk_step(NCH - 2, (NCH - 2) % 4, False)
    chunk_step(NCH - 1, (NCH - 1) % 4, False)
    for uc in range(2):
        wait_scatter(uc)
    plsc.subcore_barrier()

    @pl.when(s < TILES_IO)
    def _dump_acc():
        pltpu.sync_copy(acc_sh.at[pl.ds(s * RPT, RPT)],
                        out_hbm.at[pl.ds(c * N + s * RPT, RPT)])


_edge_kernel = pl.kernel(
    _edge_body,
    out_type=jax.ShapeDtypeStruct((2 * N, D), jnp.float32),
    mesh=plsc.VectorSubcoreMesh(core_axis_name="c", subcore_axis_name="s"),
    scratch_types=[
        pltpu.VMEM_SHARED((N, D), jnp.float32),
        pltpu.VMEM((ZR, D), jnp.float32),
        pltpu.VMEM((PK,), jnp.int32),
        pltpu.VMEM((PK,), jnp.int32),
        pltpu.VMEM((PK,), jnp.int32),
        pltpu.VMEM((PK,), jnp.int32),
        pltpu.VMEM((CHUNK,), jnp.int32),
        pltpu.VMEM((CHUNK,), jnp.int32),
        pltpu.VMEM((PK, H), jnp.bfloat16),
        pltpu.VMEM((PK, H), jnp.bfloat16),
        pltpu.VMEM((PK, H), jnp.bfloat16),
        pltpu.VMEM((PK, H), jnp.bfloat16),
        pltpu.VMEM((CHUNK, D), jnp.float32),
        pltpu.VMEM((CHUNK, D), jnp.float32),
        pltpu.SemaphoreType.DMA,
        pltpu.SemaphoreType.DMA,
        pltpu.SemaphoreType.DMA,
        pltpu.SemaphoreType.DMA,
        pltpu.SemaphoreType.DMA,
        pltpu.SemaphoreType.DMA,
        pltpu.SemaphoreType.DMA,
        pltpu.SemaphoreType.DMA,
        pltpu.SemaphoreType.DMA,
        pltpu.SemaphoreType.DMA,
    ],
    compiler_params=pltpu.CompilerParams(use_tc_tiling_on_sc=False,
                                         needs_layout_passes=False),
)


def _post_body(acc_ref, snorm_ref, gamma_ref, beta_ref, out_ref):
    sn = snorm_ref[...]
    for hh in range(2):
        dnm = acc_ref[hh * N:(hh + 1) * N, 0:H]
        num = acc_ref[hh * N:(hh + 1) * N, H:D]
        dnm = jnp.where(dnm == 0.0, 1.0, dnm)
        hagg = num / dnm * sn
        mu = jnp.mean(hagg, axis=0, keepdims=True)
        xc = hagg - mu
        var = jnp.mean(xc * xc, axis=0, keepdims=True)
        g = gamma_ref[0:1, hh * H:(hh + 1) * H]
        b = beta_ref[0:1, hh * H:(hh + 1) * H]
        y = xc * (g * lax.rsqrt(var + EPS)) + b
        out_ref[:, hh * H:(hh + 1) * H] = jnp.where(y > 0, y, jnp.exp(y) - 1.0)


def kernel(h, edge_index, snorm_n, W_fc, gamma, beta):
    src = edge_index[0].astype(jnp.int32)
    dst = edge_index[1].astype(jnp.int32)
    pidx = jnp.concatenate(
        [src.reshape(-1, CHUNK), dst.reshape(-1, CHUNK)], axis=1).reshape(-1)
    w_perm = W_fc[np.concatenate([_PERM, H + _PERM])]
    z_lo, z_hi = pl.pallas_call(
        _matmul_body,
        out_shape=[jax.ShapeDtypeStruct((N, H), jnp.bfloat16),
                   jax.ShapeDtypeStruct((N, H), jnp.bfloat16)],
    )(h, w_perm)
    acc = _edge_kernel(z_lo, z_hi, pidx)
    out = pl.pallas_call(
        _post_body,
        out_shape=jax.ShapeDtypeStruct((N, D), jnp.float32),
    )(acc, snorm_n, gamma.reshape(1, D), beta.reshape(1, D))
    return out
